# baseline (speedup 1.0000x reference)
import functools

import jax
import jax.numpy as jnp
from jax import lax
from jax.experimental import pallas as pl
from jax.experimental.pallas import tpu as pltpu

_NEG = -1e30
_NBUF = 3


def _round_up(x, m):
    return ((x + m - 1) // m) * m


def _gather_loss_kernel(idx_sref, tgt_sref, emb_ref, logits_ref, part_ref,
                        emb2_scr, emb3_scr, lse2_scr, lse3_scr,
                        gbuf, copy_sem, wsem,
                        *, tile, unroll, v_pad, bt, num_tiles, nbuf,
                        lse_blk, masked):
    i = pl.program_id(0)
    base = i * tile
    slot = lax.rem(i, nbuf)
    slot_ref = gbuf.at[pl.ds(slot * tile, tile)]
    out_slab = logits_ref.at[pl.ds(pl.multiple_of(base, 8), tile), :]

    @pl.when(i == 0)
    def _():
        cp = pltpu.make_async_copy(emb_ref, emb2_scr, copy_sem)
        cp.start()
        cp.wait()

        def lse_body(b, carry):
            off = pl.multiple_of(b * lse_blk, 8)
            x = emb2_scr[pl.ds(off, lse_blk), :]
            m = jnp.max(x, axis=-1, keepdims=True)
            s = jnp.sum(jnp.exp(x - m), axis=-1, keepdims=True)
            lse2_scr[pl.ds(off, lse_blk), :] = m + jnp.log(s)
            return carry

        lax.fori_loop(0, v_pad // lse_blk, lse_body, 0)

        cp2 = pltpu.make_async_copy(emb2_scr, emb3_scr.at[:, 0, :], copy_sem)
        cp2.start()
        cp3 = pltpu.make_async_copy(lse2_scr, lse3_scr.at[:, 0, :], copy_sem)
        cp3.start()
        cp2.wait()
        cp3.wait()

    @pl.when(i >= nbuf)
    def _():
        pltpu.make_async_copy(slot_ref.at[:, 0, :], out_slab,
                              wsem.at[slot]).wait()

    lane = lax.broadcasted_iota(jnp.int32, (1, v_pad), 1)[0]

    def trip(o, carry):
        acc_sel, acc_lse = carry
        mi0 = o * unroll
        for u in range(unroll):
            mi = mi0 + u
            r = idx_sref[base + mi]
            t = tgt_sref[base + mi]
            row = emb3_scr[r, 0, :]
            slot_ref[mi, 0, :] = row
            sel = jnp.where(lane == t, row, 0.0)
            lse_v = lse3_scr[r, 0, :]
            if masked:
                valid = (base + mi) < bt
                sel = jnp.where(valid, sel, 0.0)
                lse_v = jnp.where(valid, lse_v, 0.0)
            acc_sel = acc_sel + sel
            acc_lse = acc_lse + lse_v
        return acc_sel, acc_lse

    acc_sel, acc_lse = lax.fori_loop(
        0, tile // unroll, trip,
        (jnp.zeros((v_pad,), jnp.float32), jnp.zeros((1,), jnp.float32)))

    pltpu.make_async_copy(slot_ref.at[:, 0, :], out_slab,
                          wsem.at[slot]).start()

    part_ref[0, 0, :] = acc_lse - jnp.sum(acc_sel, keepdims=True)

    @pl.when(i == num_tiles - 1)
    def _():
        for s_ in range(nbuf):
            pltpu.make_async_copy(gbuf.at[pl.ds(s_ * tile, tile), 0, :],
                                  out_slab, wsem.at[s_]).wait()


def _pad_1d(tok, bt, bt_pad):
    tok = tok.reshape(bt).astype(jnp.int32)
    if bt_pad != bt:
        tok = jnp.concatenate([tok, jnp.zeros((bt_pad - bt,), jnp.int32)])
    return tok


def kernel(emb_table, idx, targets, *, tile=512, unroll=16):
    B, T = idx.shape
    V = emb_table.shape[0]
    BT = B * T

    v_pad = _round_up(V, 128)
    tile = min(tile, _round_up(BT, 8))
    bt_pad = _round_up(BT, tile)
    num_tiles = bt_pad // tile
    nbuf = min(_NBUF, num_tiles)
    lse_blk = 256 if v_pad % 256 == 0 else 128

    if v_pad == V:
        emb_pad = emb_table.astype(jnp.float32)
    else:
        emb_pad = jnp.full((v_pad, v_pad), _NEG, dtype=jnp.float32)
        emb_pad = emb_pad.at[:V, :V].set(emb_table.astype(jnp.float32))

    idx_flat = _pad_1d(idx, BT, bt_pad)
    tgt_flat = _pad_1d(targets if targets is not None else idx, BT, bt_pad)

    body = functools.partial(_gather_loss_kernel, tile=tile, unroll=unroll,
                             v_pad=v_pad, bt=BT, num_tiles=num_tiles,
                             nbuf=nbuf, lse_blk=lse_blk,
                             masked=(bt_pad != BT))

    logits, partials = pl.pallas_call(
        body,
        out_shape=(
            jax.ShapeDtypeStruct((bt_pad, v_pad), jnp.float32),
            jax.ShapeDtypeStruct((num_tiles, 1, 1), jnp.float32),
        ),
        grid=(num_tiles,),
        in_specs=[
            pl.BlockSpec(memory_space=pltpu.SMEM),
            pl.BlockSpec(memory_space=pltpu.SMEM),
            pl.BlockSpec(memory_space=pl.ANY),
        ],
        out_specs=(
            pl.BlockSpec(memory_space=pl.ANY),
            pl.BlockSpec((1, 1, 1), lambda i: (i, 0, 0)),
        ),
        scratch_shapes=[
            pltpu.VMEM((v_pad, v_pad), jnp.float32),
            pltpu.VMEM((v_pad, 1, v_pad), jnp.float32),
            pltpu.VMEM((v_pad, 1), jnp.float32),
            pltpu.VMEM((v_pad, 1, 1), jnp.float32),
            pltpu.VMEM((nbuf * tile, 1, v_pad), jnp.float32),
            pltpu.SemaphoreType.DMA,
            pltpu.SemaphoreType.DMA((nbuf,)),
        ],
        compiler_params=pltpu.CompilerParams(
            dimension_semantics=("arbitrary",),
            vmem_limit_bytes=58 * 1024 * 1024,
        ),
        cost_estimate=pl.CostEstimate(
            flops=8 * bt_pad * v_pad,
            transcendentals=v_pad * v_pad,
            bytes_accessed=(v_pad * v_pad * 4 + bt_pad * v_pad * 4
                            + 2 * bt_pad * 4),
        ),
    )(idx_flat, tgt_flat, emb_pad)

    if bt_pad != BT or v_pad != V:
        logits = logits[:BT, :V]

    if targets is None:
        return logits.reshape(B, T, V), None

    return logits, jnp.sum(partials) / BT


# speedup vs baseline: 1.2629x; 1.2629x over previous
import functools

import jax
import jax.numpy as jnp
from jax import lax
from jax.experimental import pallas as pl
from jax.experimental.pallas import tpu as pltpu

_NEG = -1e30


def _round_up(x, m):
    return ((x + m - 1) // m) * m


def _gather_ce_kernel(idx_sref, tgt_ref, emb_ref, logits_ref, part_ref,
                      emb3_scr, copy_sem,
                      *, tile, groups_per_trip, v_pad, bt):
    i = pl.program_id(0)
    base = i * tile

    @pl.when(i == 0)
    def _():
        cp = pltpu.make_async_copy(emb_ref, emb3_scr.at[:, 0, :], copy_sem)
        cp.start()
        cp.wait()

    def trip(o, carry):
        for g in range(groups_per_trip):
            row0 = o * groups_per_trip * 8 + g * 8
            rows = [emb3_scr[idx_sref[base + row0 + k], 0, :]
                    for k in range(8)]
            x8 = jnp.stack(rows, axis=0)
            logits_ref[pl.ds(pl.multiple_of(row0, 8), 8), :] = x8
        return carry

    lax.fori_loop(0, tile // (8 * groups_per_trip), trip, 0)

    x = logits_ref[...]
    m = jnp.max(x, axis=-1, keepdims=True)
    s = jnp.sum(jnp.exp(x - m), axis=-1, keepdims=True)
    lane = lax.broadcasted_iota(jnp.int32, x.shape, 1)
    tgt_logit = jnp.sum(jnp.where(lane == tgt_ref[...], x, 0.0),
                        axis=-1, keepdims=True)
    per_ex = m + jnp.log(s) - tgt_logit
    row_id = base + lax.broadcasted_iota(jnp.int32, (tile, 1), 0)
    per_ex = jnp.where(row_id < bt, per_ex, 0.0)
    part_ref[0, 0, :] = jnp.sum(per_ex).reshape(1)


def _pad_1d(tok, bt, bt_pad):
    tok = tok.reshape(bt).astype(jnp.int32)
    if bt_pad != bt:
        tok = jnp.concatenate([tok, jnp.zeros((bt_pad - bt,), jnp.int32)])
    return tok


def kernel(emb_table, idx, targets, *, tile=1024, groups_per_trip=8):
    B, T = idx.shape
    V = emb_table.shape[0]
    BT = B * T

    v_pad = _round_up(V, 128)
    tile = min(tile, _round_up(BT, 8))
    bt_pad = _round_up(BT, tile)
    num_tiles = bt_pad // tile

    if v_pad == V:
        emb_pad = emb_table.astype(jnp.float32)
    else:
        emb_pad = jnp.full((v_pad, v_pad), _NEG, dtype=jnp.float32)
        emb_pad = emb_pad.at[:V, :V].set(emb_table.astype(jnp.float32))

    idx_flat = _pad_1d(idx, BT, bt_pad)
    tgt_flat = _pad_1d(targets if targets is not None else idx, BT, bt_pad)
    tgt2 = tgt_flat.reshape(bt_pad, 1)

    body = functools.partial(_gather_ce_kernel, tile=tile,
                             groups_per_trip=groups_per_trip,
                             v_pad=v_pad, bt=BT)

    logits, partials = pl.pallas_call(
        body,
        out_shape=(
            jax.ShapeDtypeStruct((bt_pad, v_pad), jnp.float32),
            jax.ShapeDtypeStruct((num_tiles, 1, 1), jnp.float32),
        ),
        grid=(num_tiles,),
        in_specs=[
            pl.BlockSpec(memory_space=pltpu.SMEM),
            pl.BlockSpec((tile, 1), lambda i: (i, 0)),
            pl.BlockSpec(memory_space=pl.ANY),
        ],
        out_specs=(
            pl.BlockSpec((tile, v_pad), lambda i: (i, 0)),
            pl.BlockSpec((1, 1, 1), lambda i: (i, 0, 0)),
        ),
        scratch_shapes=[
            pltpu.VMEM((v_pad, 1, v_pad), jnp.float32),
            pltpu.SemaphoreType.DMA,
        ],
        compiler_params=pltpu.CompilerParams(
            dimension_semantics=("arbitrary",),
            vmem_limit_bytes=58 * 1024 * 1024,
        ),
        cost_estimate=pl.CostEstimate(
            flops=8 * bt_pad * v_pad,
            transcendentals=bt_pad * v_pad,
            bytes_accessed=(v_pad * v_pad * 4 + bt_pad * v_pad * 4
                            + 2 * bt_pad * 4),
        ),
    )(idx_flat, tgt2, emb_pad)

    if bt_pad != BT or v_pad != V:
        logits = logits[:BT, :V]

    if targets is None:
        return logits.reshape(B, T, V), None

    return logits, jnp.sum(partials) / BT


# speedup vs baseline: 1.2876x; 1.0196x over previous
import functools

import jax
import jax.numpy as jnp
from jax import lax
from jax.experimental import pallas as pl
from jax.experimental.pallas import tpu as pltpu

_NEG = -1e30


def _round_up(x, m):
    return ((x + m - 1) // m) * m


def _gather_ce_kernel(idx_sref, tgt_ref, emb_ref, logits_ref, part_ref,
                      emb3_scr, copy_sem,
                      *, tile, groups_per_trip, v_pad, bt):
    i = pl.program_id(0)
    base = i * tile

    @pl.when(i == 0)
    def _():
        cp = pltpu.make_async_copy(emb_ref, emb3_scr.at[:, 0, :], copy_sem)
        cp.start()
        cp.wait()

    def trip(o, carry):
        for g in range(groups_per_trip):
            row0 = o * groups_per_trip * 8 + g * 8
            rows = [emb3_scr[idx_sref[base + row0 + k], 0, :]
                    for k in range(8)]
            x8 = jnp.stack(rows, axis=0)
            logits_ref[pl.ds(pl.multiple_of(row0, 8), 8), :] = x8
        return carry

    lax.fori_loop(0, tile // (8 * groups_per_trip), trip, 0)

    x = logits_ref[...]
    m = jnp.max(x, axis=-1, keepdims=True)
    s = jnp.sum(jnp.exp(x - m), axis=-1, keepdims=True)
    lane = lax.broadcasted_iota(jnp.int32, x.shape, 1)
    tgt_logit = jnp.sum(jnp.where(lane == tgt_ref[...], x, 0.0),
                        axis=-1, keepdims=True)
    per_ex = m + jnp.log(s) - tgt_logit
    row_id = base + lax.broadcasted_iota(jnp.int32, (tile, 1), 0)
    per_ex = jnp.where(row_id < bt, per_ex, 0.0)
    part_ref[0, 0, :] = jnp.sum(per_ex).reshape(1)


def _pad_1d(tok, bt, bt_pad):
    tok = tok.reshape(bt).astype(jnp.int32)
    if bt_pad != bt:
        tok = jnp.concatenate([tok, jnp.zeros((bt_pad - bt,), jnp.int32)])
    return tok


def kernel(emb_table, idx, targets, *, tile=1024, groups_per_trip=4):
    B, T = idx.shape
    V = emb_table.shape[0]
    BT = B * T

    v_pad = _round_up(V, 128)
    tile = min(tile, _round_up(BT, 8))
    bt_pad = _round_up(BT, tile)
    num_tiles = bt_pad // tile

    if v_pad == V:
        emb_pad = emb_table.astype(jnp.float32)
    else:
        emb_pad = jnp.full((v_pad, v_pad), _NEG, dtype=jnp.float32)
        emb_pad = emb_pad.at[:V, :V].set(emb_table.astype(jnp.float32))

    idx_flat = _pad_1d(idx, BT, bt_pad)
    tgt_flat = _pad_1d(targets if targets is not None else idx, BT, bt_pad)
    tgt2 = tgt_flat.reshape(bt_pad, 1)

    body = functools.partial(_gather_ce_kernel, tile=tile,
                             groups_per_trip=groups_per_trip,
                             v_pad=v_pad, bt=BT)

    logits, partials = pl.pallas_call(
        body,
        out_shape=(
            jax.ShapeDtypeStruct((bt_pad, v_pad), jnp.float32),
            jax.ShapeDtypeStruct((num_tiles, 1, 1), jnp.float32),
        ),
        grid=(num_tiles,),
        in_specs=[
            pl.BlockSpec(memory_space=pltpu.SMEM),
            pl.BlockSpec((tile, 1), lambda i: (i, 0)),
            pl.BlockSpec(memory_space=pl.ANY),
        ],
        out_specs=(
            pl.BlockSpec((tile, v_pad), lambda i: (i, 0)),
            pl.BlockSpec((1, 1, 1), lambda i: (i, 0, 0)),
        ),
        scratch_shapes=[
            pltpu.VMEM((v_pad, 1, v_pad), jnp.float32),
            pltpu.SemaphoreType.DMA,
        ],
        compiler_params=pltpu.CompilerParams(
            dimension_semantics=("arbitrary",),
            vmem_limit_bytes=58 * 1024 * 1024,
        ),
        cost_estimate=pl.CostEstimate(
            flops=8 * bt_pad * v_pad,
            transcendentals=bt_pad * v_pad,
            bytes_accessed=(v_pad * v_pad * 4 + bt_pad * v_pad * 4
                            + 2 * bt_pad * 4),
        ),
    )(idx_flat, tgt2, emb_pad)

    if bt_pad != BT or v_pad != V:
        logits = logits[:BT, :V]

    if targets is None:
        return logits.reshape(B, T, V), None

    return logits, jnp.sum(partials) / BT
